# baseline (speedup 1.0000x reference)
import dataclasses
import functools

import jax
import jax.numpy as jnp
from jax import lax
from jax.experimental import pallas as pl
from jax.experimental.pallas import tpu as pltpu
from jax.experimental.pallas import tpu_sc as plsc

N = 10000
E = 5000
NNZ = 320000
D = 128
NH = 128
B = 1024
DCF = 16
NCLS = 10
LH = 2 * NH // 3

EPAD = 5120
NPAD = 10240
CH = 128
CH1 = 64
NW = 32
NNZP = 327680
CHUNKS = NNZP // CH
CHUNKS1 = NNZP // CH1
CPT = CHUNKS1 // NW
NHALF = NPAD // 2
XUP = NHALF + 128
XTR = XUP // 16
EACC = 5120
EDUM = EACC - 1

POOL_BLK = 1024
POOL_STEPS = NPAD // POOL_BLK

_PREC = jax.lax.Precision.HIGHEST
_MESH = plsc.VectorSubcoreMesh(core_axis_name="c", subcore_axis_name="s")

_SC_PARAMS = pltpu.CompilerParams()
if "needs_layout_passes" in pltpu.CompilerParams.__dataclass_fields__:
    _SC_PARAMS = dataclasses.replace(_SC_PARAMS, needs_layout_passes=False)


@functools.partial(
    pl.kernel,
    out_type=[
        jax.ShapeDtypeStruct((2, EACC, D), jnp.float32),
        jax.ShapeDtypeStruct((NW, EPAD), jnp.float32),
        jax.ShapeDtypeStruct((NW, NPAD), jnp.float32),
        jax.ShapeDtypeStruct((NNZP, 16), jnp.float32),
    ],
    mesh=_MESH,
    compiler_params=_SC_PARAMS,
    scratch_types=[
        pltpu.VMEM((CH1,), jnp.int32),
        pltpu.VMEM((CH1,), jnp.int32),
        pltpu.VMEM((CH1,), jnp.int32),
        pltpu.VMEM((CH1,), jnp.int32),
        pltpu.VMEM((CH1, D), jnp.float32),
        pltpu.VMEM((CH1, D), jnp.float32),
        pltpu.VMEM((CH1, D), jnp.float32),
        pltpu.VMEM((CH1, D), jnp.float32),
        pltpu.VMEM((CH1, D), jnp.float32),
        pltpu.VMEM((CH1, 16), jnp.float32),
        pltpu.VMEM((EPAD,), jnp.float32),
        pltpu.VMEM((NPAD,), jnp.float32),
        pltpu.VMEM_SHARED((EACC, D), jnp.float32),
        pltpu.SemaphoreType.DMA,
        pltpu.SemaphoreType.DMA,
    ],
)
def _sc_pass1(uep_hbm, xp_hbm, ei_hbm, ni_hbm, zacc_hbm, z1d_hbm,
              acc_out, se_out, sn_out, w_out,
              ei_a, ni_a, ei_b, ni_b, ue_a, xr_a, ue_b, xr_b, val_v,
              w_v, se_t, sn_t, acc_sh, sem_a, sem_b):
    c = lax.axis_index("c")
    s = lax.axis_index("s")
    wid = s * 2 + c
    first = wid * CPT
    last = first + CPT - 1

    pltpu.sync_copy(zacc_hbm, acc_sh.at[pl.ds(s * (EACC // 16), EACC // 16)])
    pltpu.sync_copy(z1d_hbm.at[pl.ds(0, EPAD)], se_t)
    pltpu.sync_copy(z1d_hbm, sn_t)
    plsc.subcore_barrier()

    lane0 = lax.iota(jnp.int32, 16) == 0

    def fetch(t, ei_v, ni_v, ue_v, xr_v, sem):
        pltpu.sync_copy(ei_hbm.at[pl.ds(t * CH1, CH1)], ei_v)
        pltpu.sync_copy(ni_hbm.at[pl.ds(t * CH1, CH1)], ni_v)
        g1 = pltpu.async_copy(uep_hbm.at[ei_v], ue_v, sem)
        g2 = pltpu.async_copy(xp_hbm.at[ni_v], xr_v, sem)
        return g1, g2

    def process(t, ei_v, ni_v, ue_v, xr_v):
        @plsc.parallel_loop(0, CH1 // 16, unroll=2)
        def _groups(g):
            ev16 = ei_v[pl.ds(g * 16, 16)]
            nv16 = ni_v[pl.ds(g * 16, 16)]
            for i in range(16):
                p = g * 16 + i
                acc = ue_v[p, pl.ds(0, 16)] * xr_v[p, pl.ds(0, 16)]
                for j in range(1, 8):
                    acc = acc + ue_v[p, pl.ds(16 * j, 16)] * xr_v[p, pl.ds(16 * j, 16)]
                pe = jnp.sum(acc)
                pe = jnp.where(pe >= 0.0, pe, 0.2 * pe)
                wv = jnp.exp(jnp.full((16,), pe, jnp.float32))
                w_v[p, pl.ds(0, 16)] = wv
                for j in range(8):
                    val_v[p, pl.ds(16 * j, 16)] = wv * xr_v[p, pl.ds(16 * j, 16)]
                eidx = jnp.full((16,), ev16[i], jnp.int32)
                nidx = jnp.full((16,), nv16[i], jnp.int32)
                plsc.addupdate_scatter(se_t, [eidx], wv, mask=lane0)
                plsc.addupdate_scatter(sn_t, [nidx], wv, mask=lane0)

        pltpu.sync_copy(val_v, acc_sh.at[ei_v], add=True)
        pltpu.sync_copy(w_v, w_out.at[pl.ds(t * CH1, CH1)])

    ga = fetch(first, ei_a, ni_a, ue_a, xr_a, sem_a)

    @pl.loop(0, CPT // 2)
    def _chunks(u):
        t0 = first + 2 * u
        for g in ga:
            g.wait()
        gb = fetch(t0 + 1, ei_b, ni_b, ue_b, xr_b, sem_b)
        process(t0, ei_a, ni_a, ue_a, xr_a)
        for g in gb:
            g.wait()
        ga2 = fetch(jnp.minimum(t0 + 2, last), ei_a, ni_a, ue_a, xr_a, sem_a)
        process(t0 + 1, ei_b, ni_b, ue_b, xr_b)

    for g in ga:
        g.wait()

    plsc.subcore_barrier()

    eslc = pl.ds(s * (EACC // 16), EACC // 16)
    pltpu.sync_copy(acc_sh.at[eslc], acc_out.at[c, eslc])
    pltpu.sync_copy(se_t, se_out.at[wid])
    pltpu.sync_copy(sn_t, sn_out.at[wid])


CPT2 = CHUNKS // 16


@functools.partial(
    pl.kernel,
    out_type=jax.ShapeDtypeStruct((2, XUP, D), jnp.float32),
    mesh=_MESH,
    compiler_params=_SC_PARAMS,
    scratch_types=[
        pltpu.VMEM((CH,), jnp.int32),
        pltpu.VMEM((CH,), jnp.int32),
        pltpu.VMEM((CH,), jnp.int32),
        pltpu.VMEM((CH,), jnp.int32),
        pltpu.VMEM((CH,), jnp.int32),
        pltpu.VMEM((CH, D), jnp.float32),
        pltpu.VMEM((CH, D), jnp.float32),
        pltpu.VMEM((CH, D), jnp.float32),
        pltpu.VMEM((CH, 16), jnp.float32),
        pltpu.VMEM((CH, 16), jnp.float32),
        pltpu.VMEM_SHARED((XUP, D), jnp.float32),
        pltpu.SemaphoreType.DMA,
        pltpu.SemaphoreType.DMA,
    ],
)
def _sc_pass2(xeo_hbm, ei_hbm, ni_hbm, w_hbm, zxu_hbm, xu_out,
              ei_a, ni_a, ei_b, ni_b, ni2_v, xe_a, xe_b, xval_v, w_a, w_b,
              xu_sh, sem_a, sem_b):
    c = lax.axis_index("c")
    s = lax.axis_index("s")
    offs = c * NHALF
    first = s * CPT2
    last = first + CPT2 - 1

    pltpu.sync_copy(zxu_hbm, xu_sh.at[pl.ds(s * XTR, XTR)])
    plsc.subcore_barrier()

    trash = jnp.full((16,), NHALF, jnp.int32)

    def fetch(t, ei_v, ni_v, xe_v, w_v, sem):
        pltpu.sync_copy(ei_hbm.at[pl.ds(t * CH, CH)], ei_v)
        pltpu.sync_copy(ni_hbm.at[pl.ds(t * CH, CH)], ni_v)
        g1 = pltpu.async_copy(xeo_hbm.at[ei_v], xe_v, sem)
        g2 = pltpu.async_copy(w_hbm.at[pl.ds(t * CH, CH)], w_v, sem)
        return g1, g2

    def process(ni_v, xe_v, w_v):
        @plsc.parallel_loop(0, CH // 16, unroll=2)
        def _groups(g):
            nv16 = ni_v[pl.ds(g * 16, 16)]
            lidx = nv16 - offs
            ok = (lidx >= 0) & (lidx < NHALF)
            ni2_v[pl.ds(g * 16, 16)] = jnp.where(ok, lidx, trash)
            for i in range(16):
                p = g * 16 + i
                wv = w_v[p, pl.ds(0, 16)]
                for j in range(8):
                    xval_v[p, pl.ds(16 * j, 16)] = wv * xe_v[p, pl.ds(16 * j, 16)]

        pltpu.sync_copy(xval_v, xu_sh.at[ni2_v], add=True)

    ga = fetch(first, ei_a, ni_a, xe_a, w_a, sem_a)

    @pl.loop(0, CPT2 // 2)
    def _chunks(u):
        t0 = first + 2 * u
        for g in ga:
            g.wait()
        gb = fetch(t0 + 1, ei_b, ni_b, xe_b, w_b, sem_b)
        process(ni_a, xe_a, w_a)
        for g in gb:
            g.wait()
        ga2 = fetch(jnp.minimum(t0 + 2, last), ei_a, ni_a, xe_a, w_a, sem_a)
        process(ni_b, xe_b, w_b)

    for g in ga:
        g.wait()

    plsc.subcore_barrier()

    nslc = pl.ds(s * XTR, XTR)
    pltpu.sync_copy(xu_sh.at[nslc], xu_out.at[c, nslc])


def _lin_body(x_ref, w_ref, b_ref, o_ref):
    o_ref[...] = (jnp.dot(x_ref[...], w_ref[...], precision=_PREC,
                          preferred_element_type=jnp.float32) + b_ref[...])


def _lin(x, w, b):
    rows = x.shape[0]
    return pl.pallas_call(
        _lin_body,
        grid=(rows // 1024,),
        in_specs=[
            pl.BlockSpec((1024, D), lambda i: (i, 0)),
            pl.BlockSpec((D, NH), lambda i: (0, 0)),
            pl.BlockSpec((1, NH), lambda i: (0, 0)),
        ],
        out_specs=pl.BlockSpec((1024, NH), lambda i: (i, 0)),
        out_shape=jax.ShapeDtypeStruct((rows, NH), jnp.float32),
    )(x, w, b)


def _norm_e_body(x0_ref, x1_ref, s_ref, o_ref):
    ssum = jnp.sum(s_ref[...], axis=0)[:, None]
    o_ref[...] = (x0_ref[0] + x1_ref[0]) / (ssum + 1e-9)


def _norm_e(acc, ssum):
    return pl.pallas_call(
        _norm_e_body,
        grid=((EACC + 1023) // 1024,),
        in_specs=[
            pl.BlockSpec((1, 1024, D), lambda i: (0, i, 0)),
            pl.BlockSpec((1, 1024, D), lambda i: (1, i, 0)),
            pl.BlockSpec((NW, 1024), lambda i: (0, i)),
        ],
        out_specs=pl.BlockSpec((1024, D), lambda i: (i, 0)),
        out_shape=jax.ShapeDtypeStruct((EACC, D), jnp.float32),
    )(acc, acc, ssum)


def _norm_n_body(x_ref, s_ref, o_ref):
    ssum = jnp.sum(s_ref[...], axis=0)[:, None]
    o_ref[...] = x_ref[0] / (ssum + 1e-9)


def _norm_n(xu, ssum):
    nblk = NHALF // 1024
    return pl.pallas_call(
        _norm_n_body,
        grid=(NPAD // 1024,),
        in_specs=[
            pl.BlockSpec((1, 1024, D), lambda i: (i // nblk, i % nblk, 0)),
            pl.BlockSpec((NW, 1024), lambda i: (0, i)),
        ],
        out_specs=pl.BlockSpec((1024, D), lambda i: (i, 0)),
        out_shape=jax.ShapeDtypeStruct((NPAD, D), jnp.float32),
    )(xu, ssum)


def _pool_head_body(x2_ref, sgs_ref, cf_ref, Wa_ref, va_ref, Wf_ref, bf_ref,
                    Wf2_ref, bf2_ref, Wf3_ref, bf3_ref,
                    out_ref, xsg_ref,
                    num_acc, den_acc, col_acc):
    j = pl.program_id(0)

    @pl.when(j == 0)
    def _init():
        num_acc[...] = jnp.zeros_like(num_acc)
        den_acc[...] = jnp.zeros_like(den_acc)
        col_acc[...] = jnp.zeros_like(col_acc)

    x2b = x2_ref[...]
    sgsb = sgs_ref[...]
    sb = jnp.dot(jnp.tanh(jnp.dot(x2b, Wa_ref[...], precision=_PREC,
                                  preferred_element_type=jnp.float32)),
                 va_ref[...], precision=_PREC,
                 preferred_element_type=jnp.float32)
    es = jnp.exp(sb)
    y2 = x2b * es
    num_acc[...] += jnp.dot(sgsb, y2, precision=_PREC,
                            preferred_element_type=jnp.float32)
    den_acc[...] += jnp.dot(sgsb, es, precision=_PREC,
                            preferred_element_type=jnp.float32)
    rowid = lax.broadcasted_iota(jnp.int32, (POOL_BLK, 1), 0) + j * POOL_BLK
    col_acc[...] += jnp.sum(jnp.where(rowid < N, x2b, 0.0), axis=0,
                            keepdims=True)

    @pl.when(j == POOL_STEPS - 1)
    def _final():
        den = den_acc[...]
        mean = col_acc[...] / N
        xsg = jnp.where(den > 0, num_acc[...] / jnp.where(den > 0, den, 1.0),
                        mean)
        xsg_ref[...] = xsg
        hcat = jnp.concatenate([xsg, cf_ref[...]], axis=1)
        h = jnp.maximum(jnp.dot(hcat, Wf_ref[...], precision=_PREC,
                                preferred_element_type=jnp.float32)
                        + bf_ref[...], 0.0)
        h = jnp.maximum(jnp.dot(h, Wf2_ref[...], precision=_PREC,
                                preferred_element_type=jnp.float32)
                        + bf2_ref[...], 0.0)
        out_ref[...] = jnp.dot(h, Wf3_ref[...], precision=_PREC,
                               preferred_element_type=jnp.float32) + bf3_ref[...]


def _pool_head(x2p, sgsp, cf, Wa, va, Wf, bf, Wf2, bf2, Wf3, bf3):
    full = lambda shape: pl.BlockSpec(shape, lambda j: (0,) * len(shape))
    out, xsg = pl.pallas_call(
        _pool_head_body,
        grid=(POOL_STEPS,),
        in_specs=[
            pl.BlockSpec((POOL_BLK, D), lambda j: (j, 0)),
            pl.BlockSpec((B, POOL_BLK), lambda j: (0, j)),
            full((B, DCF)),
            full((NH, NH)),
            full((NH, 1)),
            full((NH + DCF, LH)),
            full((LH,)),
            full((LH, LH)),
            full((LH,)),
            full((LH, NCLS)),
            full((NCLS,)),
        ],
        out_specs=[
            pl.BlockSpec((B, NCLS), lambda j: (0, 0)),
            pl.BlockSpec((B, NH), lambda j: (0, 0)),
        ],
        out_shape=[
            jax.ShapeDtypeStruct((B, NCLS), jnp.float32),
            jax.ShapeDtypeStruct((B, NH), jnp.float32),
        ],
        scratch_shapes=[
            pltpu.VMEM((B, NH), jnp.float32),
            pltpu.VMEM((B, 1), jnp.float32),
            pltpu.VMEM((1, D), jnp.float32),
        ],
    )(x2p, sgsp, cf, Wa, va, Wf, bf, Wf2, bf2, Wf3, bf3)
    return out, xsg


def kernel(x, xe, sgs, cf, W1, b1, a1, W2, b2, a2, Wa, va, Wf, bf, Wf2, bf2,
           Wf3, bf3, pair):
    f32 = jnp.float32
    xpad = jnp.zeros((NPAD, D), f32).at[:N].set(x)
    xepad = jnp.zeros((EPAD, D), f32).at[:E].set(xe)
    npad = NNZP - NNZ
    eip = jnp.concatenate([pair[0], jnp.full((npad,), EDUM, jnp.int32)])
    nip = jnp.concatenate([pair[1], jnp.full((npad,), NPAD - 1, jnp.int32)])
    sgsp = jnp.zeros((B, NPAD), f32).at[:, :N].set(sgs)
    zacc = jnp.zeros((EACC // 16, D), f32)
    z1d = jnp.zeros((NPAD,), f32)
    zxu = jnp.zeros((XTR, D), f32)

    def layer(xin, xein, W, b, a):
        xp = _lin(xin, W, b.reshape(1, NH))
        uep = _lin(xein, W * a[:, 0][None, :], (b * a[:, 0]).reshape(1, NH))
        acc, se, sn, w = _sc_pass1(uep, xp, eip, nip, zacc, z1d)
        xeo = _norm_e(acc, se)
        xu = _sc_pass2(xeo, eip, nip, w, zxu)
        xo = _norm_n(xu, sn)
        return xo, xeo

    x1, xe1 = layer(xpad, xepad, W1, b1, a1)
    x2, xe2p = layer(x1, xe1, W2, b2, a2)
    out, xsg = _pool_head(x2, sgsp, cf, Wa, va, Wf, bf, Wf2, bf2, Wf3, bf3)
    return (out, xsg, out, xe2p[:E])


# speedup vs baseline: 1.1486x; 1.1486x over previous
import dataclasses
import functools

import jax
import jax.numpy as jnp
from jax import lax
from jax.experimental import pallas as pl
from jax.experimental.pallas import tpu as pltpu
from jax.experimental.pallas import tpu_sc as plsc

N = 10000
E = 5000
NNZ = 320000
D = 128
NH = 128
B = 1024
DCF = 16
NCLS = 10
LH = 2 * NH // 3

EPAD = 5120
NPAD = 10240
CH = 128
CH1 = 64
NW = 32
NNZP = 327680
CHUNKS = NNZP // CH
CHUNKS1 = NNZP // CH1
CPT = CHUNKS1 // NW
NHALF = NPAD // 2
XUP = NHALF + 128
XTR = XUP // 16
EACC = 5120
EDUM = EACC - 1

POOL_BLK = 1024
POOL_STEPS = NPAD // POOL_BLK

_PREC = jax.lax.Precision.HIGHEST
_MESH = plsc.VectorSubcoreMesh(core_axis_name="c", subcore_axis_name="s")

_SC_PARAMS = pltpu.CompilerParams()
if "needs_layout_passes" in pltpu.CompilerParams.__dataclass_fields__:
    _SC_PARAMS = dataclasses.replace(_SC_PARAMS, needs_layout_passes=False)


@functools.partial(
    pl.kernel,
    out_type=[
        jax.ShapeDtypeStruct((2, EACC, D), jnp.float32),
        jax.ShapeDtypeStruct((NW, EPAD), jnp.float32),
        jax.ShapeDtypeStruct((NW, NPAD), jnp.float32),
        jax.ShapeDtypeStruct((NNZP, 16), jnp.float32),
    ],
    mesh=_MESH,
    compiler_params=_SC_PARAMS,
    scratch_types=[
        pltpu.VMEM((CH1,), jnp.int32),
        pltpu.VMEM((CH1,), jnp.int32),
        pltpu.VMEM((CH1,), jnp.int32),
        pltpu.VMEM((CH1,), jnp.int32),
        pltpu.VMEM((CH1, D), jnp.float32),
        pltpu.VMEM((CH1, D), jnp.float32),
        pltpu.VMEM((CH1, D), jnp.float32),
        pltpu.VMEM((CH1, D), jnp.float32),
        pltpu.VMEM((CH1, D), jnp.float32),
        pltpu.VMEM((CH1, 16), jnp.float32),
        pltpu.VMEM((EPAD,), jnp.float32),
        pltpu.VMEM((NPAD,), jnp.float32),
        pltpu.VMEM_SHARED((EACC, D), jnp.float32),
        pltpu.SemaphoreType.DMA,
        pltpu.SemaphoreType.DMA,
    ],
)
def _sc_pass1(uep_hbm, xp_hbm, ei_hbm, ni_hbm, zacc_hbm, z1d_hbm,
              acc_out, se_out, sn_out, w_out,
              ei_a, ni_a, ei_b, ni_b, ue_a, xr_a, ue_b, xr_b, val_v,
              w_v, se_t, sn_t, acc_sh, sem_a, sem_b):
    c = lax.axis_index("c")
    s = lax.axis_index("s")
    wid = s * 2 + c
    first = wid * CPT
    last = first + CPT - 1

    pltpu.sync_copy(zacc_hbm, acc_sh.at[pl.ds(s * (EACC // 16), EACC // 16)])
    pltpu.sync_copy(z1d_hbm.at[pl.ds(0, EPAD)], se_t)
    pltpu.sync_copy(z1d_hbm, sn_t)
    plsc.subcore_barrier()

    lane0 = lax.iota(jnp.int32, 16) == 0

    def fetch(t, ei_v, ni_v, ue_v, xr_v, sem):
        pltpu.sync_copy(ei_hbm.at[pl.ds(t * CH1, CH1)], ei_v)
        pltpu.sync_copy(ni_hbm.at[pl.ds(t * CH1, CH1)], ni_v)
        g1 = pltpu.async_copy(uep_hbm.at[ei_v], ue_v, sem)
        g2 = pltpu.async_copy(xp_hbm.at[ni_v], xr_v, sem)
        return g1, g2

    def process(t, ei_v, ni_v, ue_v, xr_v):
        @pl.loop(0, CH1 // 16)
        def _groups(g):
            ev16 = ei_v[pl.ds(g * 16, 16)]
            nv16 = ni_v[pl.ds(g * 16, 16)]
            for i in range(16):
                p = g * 16 + i
                acc = ue_v[p, pl.ds(0, 16)] * xr_v[p, pl.ds(0, 16)]
                for j in range(1, 8):
                    acc = acc + ue_v[p, pl.ds(16 * j, 16)] * xr_v[p, pl.ds(16 * j, 16)]
                pe = jnp.sum(acc)
                pe = jnp.where(pe >= 0.0, pe, 0.2 * pe)
                wv = jnp.exp(jnp.full((16,), pe, jnp.float32))
                w_v[p, pl.ds(0, 16)] = wv
                for j in range(8):
                    val_v[p, pl.ds(16 * j, 16)] = wv * xr_v[p, pl.ds(16 * j, 16)]
                eidx = jnp.full((16,), ev16[i], jnp.int32)
                nidx = jnp.full((16,), nv16[i], jnp.int32)
                plsc.addupdate_scatter(se_t, [eidx], wv, mask=lane0)
                plsc.addupdate_scatter(sn_t, [nidx], wv, mask=lane0)

        pltpu.sync_copy(val_v, acc_sh.at[ei_v], add=True)
        pltpu.sync_copy(w_v, w_out.at[pl.ds(t * CH1, CH1)])

    ga = fetch(first, ei_a, ni_a, ue_a, xr_a, sem_a)

    @pl.loop(0, CPT // 2)
    def _chunks(u):
        t0 = first + 2 * u
        for g in ga:
            g.wait()
        gb = fetch(t0 + 1, ei_b, ni_b, ue_b, xr_b, sem_b)
        process(t0, ei_a, ni_a, ue_a, xr_a)
        for g in gb:
            g.wait()
        ga2 = fetch(jnp.minimum(t0 + 2, last), ei_a, ni_a, ue_a, xr_a, sem_a)
        process(t0 + 1, ei_b, ni_b, ue_b, xr_b)

    for g in ga:
        g.wait()

    plsc.subcore_barrier()

    eslc = pl.ds(s * (EACC // 16), EACC // 16)
    pltpu.sync_copy(acc_sh.at[eslc], acc_out.at[c, eslc])
    pltpu.sync_copy(se_t, se_out.at[wid])
    pltpu.sync_copy(sn_t, sn_out.at[wid])


CPT2 = CHUNKS // 16


@functools.partial(
    pl.kernel,
    out_type=jax.ShapeDtypeStruct((2, XUP, D), jnp.float32),
    mesh=_MESH,
    compiler_params=_SC_PARAMS,
    scratch_types=[
        pltpu.VMEM((CH,), jnp.int32),
        pltpu.VMEM((CH,), jnp.int32),
        pltpu.VMEM((CH,), jnp.int32),
        pltpu.VMEM((CH,), jnp.int32),
        pltpu.VMEM((CH,), jnp.int32),
        pltpu.VMEM((CH, D), jnp.float32),
        pltpu.VMEM((CH, D), jnp.float32),
        pltpu.VMEM((CH, D), jnp.float32),
        pltpu.VMEM((CH, 16), jnp.float32),
        pltpu.VMEM((CH, 16), jnp.float32),
        pltpu.VMEM_SHARED((XUP, D), jnp.float32),
        pltpu.SemaphoreType.DMA,
        pltpu.SemaphoreType.DMA,
    ],
)
def _sc_pass2(xeo_hbm, ei_hbm, ni_hbm, w_hbm, zxu_hbm, xu_out,
              ei_a, ni_a, ei_b, ni_b, ni2_v, xe_a, xe_b, xval_v, w_a, w_b,
              xu_sh, sem_a, sem_b):
    c = lax.axis_index("c")
    s = lax.axis_index("s")
    offs = c * NHALF
    first = s * CPT2
    last = first + CPT2 - 1

    pltpu.sync_copy(zxu_hbm, xu_sh.at[pl.ds(s * XTR, XTR)])
    plsc.subcore_barrier()

    trash = jnp.full((16,), NHALF, jnp.int32)

    def fetch(t, ei_v, ni_v, xe_v, w_v, sem):
        pltpu.sync_copy(ei_hbm.at[pl.ds(t * CH, CH)], ei_v)
        pltpu.sync_copy(ni_hbm.at[pl.ds(t * CH, CH)], ni_v)
        g1 = pltpu.async_copy(xeo_hbm.at[ei_v], xe_v, sem)
        g2 = pltpu.async_copy(w_hbm.at[pl.ds(t * CH, CH)], w_v, sem)
        return g1, g2

    def process(ni_v, xe_v, w_v):
        @pl.loop(0, CH // 16)
        def _groups(g):
            nv16 = ni_v[pl.ds(g * 16, 16)]
            lidx = nv16 - offs
            ok = (lidx >= 0) & (lidx < NHALF)
            ni2_v[pl.ds(g * 16, 16)] = jnp.where(ok, lidx, trash)
            for i in range(16):
                p = g * 16 + i
                wv = w_v[p, pl.ds(0, 16)]
                for j in range(8):
                    xval_v[p, pl.ds(16 * j, 16)] = wv * xe_v[p, pl.ds(16 * j, 16)]

        pltpu.sync_copy(xval_v, xu_sh.at[ni2_v], add=True)

    ga = fetch(first, ei_a, ni_a, xe_a, w_a, sem_a)

    @pl.loop(0, CPT2 // 2)
    def _chunks(u):
        t0 = first + 2 * u
        for g in ga:
            g.wait()
        gb = fetch(t0 + 1, ei_b, ni_b, xe_b, w_b, sem_b)
        process(ni_a, xe_a, w_a)
        for g in gb:
            g.wait()
        ga2 = fetch(jnp.minimum(t0 + 2, last), ei_a, ni_a, xe_a, w_a, sem_a)
        process(ni_b, xe_b, w_b)

    for g in ga:
        g.wait()

    plsc.subcore_barrier()

    nslc = pl.ds(s * XTR, XTR)
    pltpu.sync_copy(xu_sh.at[nslc], xu_out.at[c, nslc])


def _lin_body(x_ref, w_ref, b_ref, o_ref):
    o_ref[...] = (jnp.dot(x_ref[...], w_ref[...], precision=_PREC,
                          preferred_element_type=jnp.float32) + b_ref[...])


def _lin(x, w, b):
    rows = x.shape[0]
    return pl.pallas_call(
        _lin_body,
        grid=(rows // 1024,),
        in_specs=[
            pl.BlockSpec((1024, D), lambda i: (i, 0)),
            pl.BlockSpec((D, NH), lambda i: (0, 0)),
            pl.BlockSpec((1, NH), lambda i: (0, 0)),
        ],
        out_specs=pl.BlockSpec((1024, NH), lambda i: (i, 0)),
        out_shape=jax.ShapeDtypeStruct((rows, NH), jnp.float32),
    )(x, w, b)


def _norm_e_body(x0_ref, x1_ref, s_ref, o_ref):
    ssum = jnp.sum(s_ref[...], axis=0)[:, None]
    o_ref[...] = (x0_ref[0] + x1_ref[0]) / (ssum + 1e-9)


def _norm_e(acc, ssum):
    return pl.pallas_call(
        _norm_e_body,
        grid=((EACC + 1023) // 1024,),
        in_specs=[
            pl.BlockSpec((1, 1024, D), lambda i: (0, i, 0)),
            pl.BlockSpec((1, 1024, D), lambda i: (1, i, 0)),
            pl.BlockSpec((NW, 1024), lambda i: (0, i)),
        ],
        out_specs=pl.BlockSpec((1024, D), lambda i: (i, 0)),
        out_shape=jax.ShapeDtypeStruct((EACC, D), jnp.float32),
    )(acc, acc, ssum)


def _norm_n_body(x_ref, s_ref, o_ref):
    ssum = jnp.sum(s_ref[...], axis=0)[:, None]
    o_ref[...] = x_ref[0] / (ssum + 1e-9)


def _norm_n(xu, ssum):
    nblk = NHALF // 1024
    return pl.pallas_call(
        _norm_n_body,
        grid=(NPAD // 1024,),
        in_specs=[
            pl.BlockSpec((1, 1024, D), lambda i: (i // nblk, i % nblk, 0)),
            pl.BlockSpec((NW, 1024), lambda i: (0, i)),
        ],
        out_specs=pl.BlockSpec((1024, D), lambda i: (i, 0)),
        out_shape=jax.ShapeDtypeStruct((NPAD, D), jnp.float32),
    )(xu, ssum)


def _pool_head_body(x2_ref, sgs_ref, cf_ref, Wa_ref, va_ref, Wf_ref, bf_ref,
                    Wf2_ref, bf2_ref, Wf3_ref, bf3_ref,
                    out_ref, xsg_ref,
                    num_acc, den_acc, col_acc):
    j = pl.program_id(0)

    @pl.when(j == 0)
    def _init():
        num_acc[...] = jnp.zeros_like(num_acc)
        den_acc[...] = jnp.zeros_like(den_acc)
        col_acc[...] = jnp.zeros_like(col_acc)

    x2b = x2_ref[...]
    sgsb = sgs_ref[...]
    sb = jnp.dot(jnp.tanh(jnp.dot(x2b, Wa_ref[...], precision=_PREC,
                                  preferred_element_type=jnp.float32)),
                 va_ref[...], precision=_PREC,
                 preferred_element_type=jnp.float32)
    es = jnp.exp(sb)
    y2 = x2b * es
    num_acc[...] += jnp.dot(sgsb, y2, precision=_PREC,
                            preferred_element_type=jnp.float32)
    den_acc[...] += jnp.dot(sgsb, es, precision=_PREC,
                            preferred_element_type=jnp.float32)
    rowid = lax.broadcasted_iota(jnp.int32, (POOL_BLK, 1), 0) + j * POOL_BLK
    col_acc[...] += jnp.sum(jnp.where(rowid < N, x2b, 0.0), axis=0,
                            keepdims=True)

    @pl.when(j == POOL_STEPS - 1)
    def _final():
        den = den_acc[...]
        mean = col_acc[...] / N
        xsg = jnp.where(den > 0, num_acc[...] / jnp.where(den > 0, den, 1.0),
                        mean)
        xsg_ref[...] = xsg
        hcat = jnp.concatenate([xsg, cf_ref[...]], axis=1)
        h = jnp.maximum(jnp.dot(hcat, Wf_ref[...], precision=_PREC,
                                preferred_element_type=jnp.float32)
                        + bf_ref[...], 0.0)
        h = jnp.maximum(jnp.dot(h, Wf2_ref[...], precision=_PREC,
                                preferred_element_type=jnp.float32)
                        + bf2_ref[...], 0.0)
        out_ref[...] = jnp.dot(h, Wf3_ref[...], precision=_PREC,
                               preferred_element_type=jnp.float32) + bf3_ref[...]


def _pool_head(x2p, sgsp, cf, Wa, va, Wf, bf, Wf2, bf2, Wf3, bf3):
    full = lambda shape: pl.BlockSpec(shape, lambda j: (0,) * len(shape))
    out, xsg = pl.pallas_call(
        _pool_head_body,
        grid=(POOL_STEPS,),
        in_specs=[
            pl.BlockSpec((POOL_BLK, D), lambda j: (j, 0)),
            pl.BlockSpec((B, POOL_BLK), lambda j: (0, j)),
            full((B, DCF)),
            full((NH, NH)),
            full((NH, 1)),
            full((NH + DCF, LH)),
            full((LH,)),
            full((LH, LH)),
            full((LH,)),
            full((LH, NCLS)),
            full((NCLS,)),
        ],
        out_specs=[
            pl.BlockSpec((B, NCLS), lambda j: (0, 0)),
            pl.BlockSpec((B, NH), lambda j: (0, 0)),
        ],
        out_shape=[
            jax.ShapeDtypeStruct((B, NCLS), jnp.float32),
            jax.ShapeDtypeStruct((B, NH), jnp.float32),
        ],
        scratch_shapes=[
            pltpu.VMEM((B, NH), jnp.float32),
            pltpu.VMEM((B, 1), jnp.float32),
            pltpu.VMEM((1, D), jnp.float32),
        ],
    )(x2p, sgsp, cf, Wa, va, Wf, bf, Wf2, bf2, Wf3, bf3)
    return out, xsg


def kernel(x, xe, sgs, cf, W1, b1, a1, W2, b2, a2, Wa, va, Wf, bf, Wf2, bf2,
           Wf3, bf3, pair):
    f32 = jnp.float32
    xpad = jnp.zeros((NPAD, D), f32).at[:N].set(x)
    xepad = jnp.zeros((EPAD, D), f32).at[:E].set(xe)
    npad = NNZP - NNZ
    eip = jnp.concatenate([pair[0], jnp.full((npad,), EDUM, jnp.int32)])
    nip = jnp.concatenate([pair[1], jnp.full((npad,), NPAD - 1, jnp.int32)])
    sgsp = jnp.zeros((B, NPAD), f32).at[:, :N].set(sgs)
    zacc = jnp.zeros((EACC // 16, D), f32)
    z1d = jnp.zeros((NPAD,), f32)
    zxu = jnp.zeros((XTR, D), f32)

    def layer(xin, xein, W, b, a):
        xp = _lin(xin, W, b.reshape(1, NH))
        uep = _lin(xein, W * a[:, 0][None, :], (b * a[:, 0]).reshape(1, NH))
        acc, se, sn, w = _sc_pass1(uep, xp, eip, nip, zacc, z1d)
        xeo = _norm_e(acc, se)
        xu = _sc_pass2(xeo, eip, nip, w, zxu)
        xo = _norm_n(xu, sn)
        return xo, xeo

    x1, xe1 = layer(xpad, xepad, W1, b1, a1)
    x2, xe2p = layer(x1, xe1, W2, b2, a2)
    out, xsg = _pool_head(x2, sgsp, cf, Wa, va, Wf, bf, Wf2, bf2, Wf3, bf3)
    return (out, xsg, out, xe2p[:E])


# speedup vs baseline: 1.1728x; 1.0211x over previous
import dataclasses
import functools

import jax
import jax.numpy as jnp
from jax import lax
from jax.experimental import pallas as pl
from jax.experimental.pallas import tpu as pltpu
from jax.experimental.pallas import tpu_sc as plsc

N = 10000
E = 5000
NNZ = 320000
D = 128
NH = 128
B = 1024
DCF = 16
NCLS = 10
LH = 2 * NH // 3

EPAD = 5120
NPAD = 10240
CH = 128
CH1 = 64
NW = 32
NNZP = 327680
CHUNKS = NNZP // CH
CHUNKS1 = NNZP // CH1
CPT = CHUNKS1 // NW
NHALF = NPAD // 2
XUP = NHALF + 128
XTR = XUP // 16
EACC = 5120
EDUM = EACC - 1

POOL_BLK = 1024
POOL_STEPS = NPAD // POOL_BLK

_PREC = jax.lax.Precision.HIGHEST
_MESH = plsc.VectorSubcoreMesh(core_axis_name="c", subcore_axis_name="s")

_SC_PARAMS = pltpu.CompilerParams()
if "needs_layout_passes" in pltpu.CompilerParams.__dataclass_fields__:
    _SC_PARAMS = dataclasses.replace(_SC_PARAMS, needs_layout_passes=False)


@functools.partial(
    pl.kernel,
    out_type=[
        jax.ShapeDtypeStruct((2, EACC, D), jnp.float32),
        jax.ShapeDtypeStruct((NW, EPAD), jnp.float32),
        jax.ShapeDtypeStruct((NW, NPAD), jnp.float32),
        jax.ShapeDtypeStruct((NNZP, 16), jnp.float32),
    ],
    mesh=_MESH,
    compiler_params=_SC_PARAMS,
    scratch_types=[
        pltpu.VMEM((CH1,), jnp.int32),
        pltpu.VMEM((CH1,), jnp.int32),
        pltpu.VMEM((CH1,), jnp.int32),
        pltpu.VMEM((CH1,), jnp.int32),
        pltpu.VMEM((CH1, D), jnp.float32),
        pltpu.VMEM((CH1, D), jnp.float32),
        pltpu.VMEM((CH1, D), jnp.float32),
        pltpu.VMEM((CH1, D), jnp.float32),
        pltpu.VMEM((CH1, D), jnp.float32),
        pltpu.VMEM((CH1, 16), jnp.float32),
        pltpu.VMEM((EPAD,), jnp.float32),
        pltpu.VMEM((NPAD,), jnp.float32),
        pltpu.VMEM_SHARED((EACC, D), jnp.float32),
        pltpu.SemaphoreType.DMA,
        pltpu.SemaphoreType.DMA,
    ],
)
def _sc_pass1(uep_hbm, xp_hbm, ei_hbm, ni_hbm, zacc_hbm, z1d_hbm,
              acc_out, se_out, sn_out, w_out,
              ei_a, ni_a, ei_b, ni_b, ue_a, xr_a, ue_b, xr_b, val_v,
              w_v, se_t, sn_t, acc_sh, sem_a, sem_b):
    c = lax.axis_index("c")
    s = lax.axis_index("s")
    wid = s * 2 + c
    first = wid * CPT
    last = first + CPT - 1

    pltpu.sync_copy(zacc_hbm, acc_sh.at[pl.ds(s * (EACC // 16), EACC // 16)])
    pltpu.sync_copy(z1d_hbm.at[pl.ds(0, EPAD)], se_t)
    pltpu.sync_copy(z1d_hbm, sn_t)
    plsc.subcore_barrier()

    lane0 = lax.iota(jnp.int32, 16) == 0

    def fetch(t, ei_v, ni_v, ue_v, xr_v, sem):
        pltpu.sync_copy(ei_hbm.at[pl.ds(t * CH1, CH1)], ei_v)
        pltpu.sync_copy(ni_hbm.at[pl.ds(t * CH1, CH1)], ni_v)
        g1 = pltpu.async_copy(uep_hbm.at[ei_v], ue_v, sem)
        g2 = pltpu.async_copy(xp_hbm.at[ni_v], xr_v, sem)
        return g1, g2

    def process(t, ei_v, ni_v, ue_v, xr_v):
        @pl.loop(0, CH1 // 16)
        def _groups(g):
            ev16 = ei_v[pl.ds(g * 16, 16)]
            nv16 = ni_v[pl.ds(g * 16, 16)]
            accs = [ue_v[g * 16 + i, pl.ds(0, 16)] * xr_v[g * 16 + i, pl.ds(0, 16)]
                    for i in range(16)]
            for j in range(1, 8):
                for i in range(16):
                    p = g * 16 + i
                    accs[i] = accs[i] + (ue_v[p, pl.ds(16 * j, 16)]
                                         * xr_v[p, pl.ds(16 * j, 16)])
            wvs = []
            for i in range(16):
                pe = jnp.sum(accs[i])
                pe = jnp.where(pe >= 0.0, pe, 0.2 * pe)
                wvs.append(jnp.exp(jnp.full((16,), pe, jnp.float32)))
            for i in range(16):
                p = g * 16 + i
                w_v[p, pl.ds(0, 16)] = wvs[i]
                for j in range(8):
                    val_v[p, pl.ds(16 * j, 16)] = wvs[i] * xr_v[p, pl.ds(16 * j, 16)]
            for i in range(16):
                eidx = jnp.full((16,), ev16[i], jnp.int32)
                nidx = jnp.full((16,), nv16[i], jnp.int32)
                plsc.addupdate_scatter(se_t, [eidx], wvs[i], mask=lane0)
                plsc.addupdate_scatter(sn_t, [nidx], wvs[i], mask=lane0)

        pltpu.sync_copy(val_v, acc_sh.at[ei_v], add=True)
        pltpu.sync_copy(w_v, w_out.at[pl.ds(t * CH1, CH1)])

    ga = fetch(first, ei_a, ni_a, ue_a, xr_a, sem_a)

    @pl.loop(0, CPT // 2)
    def _chunks(u):
        t0 = first + 2 * u
        for g in ga:
            g.wait()
        gb = fetch(t0 + 1, ei_b, ni_b, ue_b, xr_b, sem_b)
        process(t0, ei_a, ni_a, ue_a, xr_a)
        for g in gb:
            g.wait()
        ga2 = fetch(jnp.minimum(t0 + 2, last), ei_a, ni_a, ue_a, xr_a, sem_a)
        process(t0 + 1, ei_b, ni_b, ue_b, xr_b)

    for g in ga:
        g.wait()

    plsc.subcore_barrier()

    eslc = pl.ds(s * (EACC // 16), EACC // 16)
    pltpu.sync_copy(acc_sh.at[eslc], acc_out.at[c, eslc])
    pltpu.sync_copy(se_t, se_out.at[wid])
    pltpu.sync_copy(sn_t, sn_out.at[wid])


CPT2 = CHUNKS // 16


@functools.partial(
    pl.kernel,
    out_type=jax.ShapeDtypeStruct((2, XUP, D), jnp.float32),
    mesh=_MESH,
    compiler_params=_SC_PARAMS,
    scratch_types=[
        pltpu.VMEM((CH,), jnp.int32),
        pltpu.VMEM((CH,), jnp.int32),
        pltpu.VMEM((CH,), jnp.int32),
        pltpu.VMEM((CH,), jnp.int32),
        pltpu.VMEM((CH,), jnp.int32),
        pltpu.VMEM((CH, D), jnp.float32),
        pltpu.VMEM((CH, D), jnp.float32),
        pltpu.VMEM((CH, D), jnp.float32),
        pltpu.VMEM((CH, 16), jnp.float32),
        pltpu.VMEM((CH, 16), jnp.float32),
        pltpu.VMEM_SHARED((XUP, D), jnp.float32),
        pltpu.SemaphoreType.DMA,
        pltpu.SemaphoreType.DMA,
    ],
)
def _sc_pass2(xeo_hbm, ei_hbm, ni_hbm, w_hbm, zxu_hbm, xu_out,
              ei_a, ni_a, ei_b, ni_b, ni2_v, xe_a, xe_b, xval_v, w_a, w_b,
              xu_sh, sem_a, sem_b):
    c = lax.axis_index("c")
    s = lax.axis_index("s")
    offs = c * NHALF
    first = s * CPT2
    last = first + CPT2 - 1

    pltpu.sync_copy(zxu_hbm, xu_sh.at[pl.ds(s * XTR, XTR)])
    plsc.subcore_barrier()

    trash = jnp.full((16,), NHALF, jnp.int32)

    def fetch(t, ei_v, ni_v, xe_v, w_v, sem):
        pltpu.sync_copy(ei_hbm.at[pl.ds(t * CH, CH)], ei_v)
        pltpu.sync_copy(ni_hbm.at[pl.ds(t * CH, CH)], ni_v)
        g1 = pltpu.async_copy(xeo_hbm.at[ei_v], xe_v, sem)
        g2 = pltpu.async_copy(w_hbm.at[pl.ds(t * CH, CH)], w_v, sem)
        return g1, g2

    def process(ni_v, xe_v, w_v):
        @pl.loop(0, CH // 16)
        def _groups(g):
            nv16 = ni_v[pl.ds(g * 16, 16)]
            lidx = nv16 - offs
            ok = (lidx >= 0) & (lidx < NHALF)
            ni2_v[pl.ds(g * 16, 16)] = jnp.where(ok, lidx, trash)
            wvs = [w_v[g * 16 + i, pl.ds(0, 16)] for i in range(16)]
            for j in range(8):
                for i in range(16):
                    p = g * 16 + i
                    xval_v[p, pl.ds(16 * j, 16)] = wvs[i] * xe_v[p, pl.ds(16 * j, 16)]

        pltpu.sync_copy(xval_v, xu_sh.at[ni2_v], add=True)

    ga = fetch(first, ei_a, ni_a, xe_a, w_a, sem_a)

    @pl.loop(0, CPT2 // 2)
    def _chunks(u):
        t0 = first + 2 * u
        for g in ga:
            g.wait()
        gb = fetch(t0 + 1, ei_b, ni_b, xe_b, w_b, sem_b)
        process(ni_a, xe_a, w_a)
        for g in gb:
            g.wait()
        ga2 = fetch(jnp.minimum(t0 + 2, last), ei_a, ni_a, xe_a, w_a, sem_a)
        process(ni_b, xe_b, w_b)

    for g in ga:
        g.wait()

    plsc.subcore_barrier()

    nslc = pl.ds(s * XTR, XTR)
    pltpu.sync_copy(xu_sh.at[nslc], xu_out.at[c, nslc])


def _lin_body(x_ref, w_ref, b_ref, o_ref):
    o_ref[...] = (jnp.dot(x_ref[...], w_ref[...], precision=_PREC,
                          preferred_element_type=jnp.float32) + b_ref[...])


def _lin(x, w, b):
    rows = x.shape[0]
    return pl.pallas_call(
        _lin_body,
        grid=(rows // 1024,),
        in_specs=[
            pl.BlockSpec((1024, D), lambda i: (i, 0)),
            pl.BlockSpec((D, NH), lambda i: (0, 0)),
            pl.BlockSpec((1, NH), lambda i: (0, 0)),
        ],
        out_specs=pl.BlockSpec((1024, NH), lambda i: (i, 0)),
        out_shape=jax.ShapeDtypeStruct((rows, NH), jnp.float32),
    )(x, w, b)


def _norm_e_body(x0_ref, x1_ref, s_ref, o_ref):
    ssum = jnp.sum(s_ref[...], axis=0)[:, None]
    o_ref[...] = (x0_ref[0] + x1_ref[0]) / (ssum + 1e-9)


def _norm_e(acc, ssum):
    return pl.pallas_call(
        _norm_e_body,
        grid=((EACC + 1023) // 1024,),
        in_specs=[
            pl.BlockSpec((1, 1024, D), lambda i: (0, i, 0)),
            pl.BlockSpec((1, 1024, D), lambda i: (1, i, 0)),
            pl.BlockSpec((NW, 1024), lambda i: (0, i)),
        ],
        out_specs=pl.BlockSpec((1024, D), lambda i: (i, 0)),
        out_shape=jax.ShapeDtypeStruct((EACC, D), jnp.float32),
    )(acc, acc, ssum)


def _norm_n_body(x_ref, s_ref, o_ref):
    ssum = jnp.sum(s_ref[...], axis=0)[:, None]
    o_ref[...] = x_ref[0] / (ssum + 1e-9)


def _norm_n(xu, ssum):
    nblk = NHALF // 1024
    return pl.pallas_call(
        _norm_n_body,
        grid=(NPAD // 1024,),
        in_specs=[
            pl.BlockSpec((1, 1024, D), lambda i: (i // nblk, i % nblk, 0)),
            pl.BlockSpec((NW, 1024), lambda i: (0, i)),
        ],
        out_specs=pl.BlockSpec((1024, D), lambda i: (i, 0)),
        out_shape=jax.ShapeDtypeStruct((NPAD, D), jnp.float32),
    )(xu, ssum)


def _pool_head_body(x2_ref, sgs_ref, cf_ref, Wa_ref, va_ref, Wf_ref, bf_ref,
                    Wf2_ref, bf2_ref, Wf3_ref, bf3_ref,
                    out_ref, xsg_ref,
                    num_acc, den_acc, col_acc):
    j = pl.program_id(0)

    @pl.when(j == 0)
    def _init():
        num_acc[...] = jnp.zeros_like(num_acc)
        den_acc[...] = jnp.zeros_like(den_acc)
        col_acc[...] = jnp.zeros_like(col_acc)

    x2b = x2_ref[...]
    sgsb = sgs_ref[...]
    sb = jnp.dot(jnp.tanh(jnp.dot(x2b, Wa_ref[...], precision=_PREC,
                                  preferred_element_type=jnp.float32)),
                 va_ref[...], precision=_PREC,
                 preferred_element_type=jnp.float32)
    es = jnp.exp(sb)
    y2 = x2b * es
    num_acc[...] += jnp.dot(sgsb, y2, precision=_PREC,
                            preferred_element_type=jnp.float32)
    den_acc[...] += jnp.dot(sgsb, es, precision=_PREC,
                            preferred_element_type=jnp.float32)
    rowid = lax.broadcasted_iota(jnp.int32, (POOL_BLK, 1), 0) + j * POOL_BLK
    col_acc[...] += jnp.sum(jnp.where(rowid < N, x2b, 0.0), axis=0,
                            keepdims=True)

    @pl.when(j == POOL_STEPS - 1)
    def _final():
        den = den_acc[...]
        mean = col_acc[...] / N
        xsg = jnp.where(den > 0, num_acc[...] / jnp.where(den > 0, den, 1.0),
                        mean)
        xsg_ref[...] = xsg
        hcat = jnp.concatenate([xsg, cf_ref[...]], axis=1)
        h = jnp.maximum(jnp.dot(hcat, Wf_ref[...], precision=_PREC,
                                preferred_element_type=jnp.float32)
                        + bf_ref[...], 0.0)
        h = jnp.maximum(jnp.dot(h, Wf2_ref[...], precision=_PREC,
                                preferred_element_type=jnp.float32)
                        + bf2_ref[...], 0.0)
        out_ref[...] = jnp.dot(h, Wf3_ref[...], precision=_PREC,
                               preferred_element_type=jnp.float32) + bf3_ref[...]


def _pool_head(x2p, sgsp, cf, Wa, va, Wf, bf, Wf2, bf2, Wf3, bf3):
    full = lambda shape: pl.BlockSpec(shape, lambda j: (0,) * len(shape))
    out, xsg = pl.pallas_call(
        _pool_head_body,
        grid=(POOL_STEPS,),
        in_specs=[
            pl.BlockSpec((POOL_BLK, D), lambda j: (j, 0)),
            pl.BlockSpec((B, POOL_BLK), lambda j: (0, j)),
            full((B, DCF)),
            full((NH, NH)),
            full((NH, 1)),
            full((NH + DCF, LH)),
            full((LH,)),
            full((LH, LH)),
            full((LH,)),
            full((LH, NCLS)),
            full((NCLS,)),
        ],
        out_specs=[
            pl.BlockSpec((B, NCLS), lambda j: (0, 0)),
            pl.BlockSpec((B, NH), lambda j: (0, 0)),
        ],
        out_shape=[
            jax.ShapeDtypeStruct((B, NCLS), jnp.float32),
            jax.ShapeDtypeStruct((B, NH), jnp.float32),
        ],
        scratch_shapes=[
            pltpu.VMEM((B, NH), jnp.float32),
            pltpu.VMEM((B, 1), jnp.float32),
            pltpu.VMEM((1, D), jnp.float32),
        ],
    )(x2p, sgsp, cf, Wa, va, Wf, bf, Wf2, bf2, Wf3, bf3)
    return out, xsg


def kernel(x, xe, sgs, cf, W1, b1, a1, W2, b2, a2, Wa, va, Wf, bf, Wf2, bf2,
           Wf3, bf3, pair):
    f32 = jnp.float32
    xpad = jnp.zeros((NPAD, D), f32).at[:N].set(x)
    xepad = jnp.zeros((EPAD, D), f32).at[:E].set(xe)
    npad = NNZP - NNZ
    eip = jnp.concatenate([pair[0], jnp.full((npad,), EDUM, jnp.int32)])
    nip = jnp.concatenate([pair[1], jnp.full((npad,), NPAD - 1, jnp.int32)])
    sgsp = jnp.zeros((B, NPAD), f32).at[:, :N].set(sgs)
    zacc = jnp.zeros((EACC // 16, D), f32)
    z1d = jnp.zeros((NPAD,), f32)
    zxu = jnp.zeros((XTR, D), f32)

    def layer(xin, xein, W, b, a):
        xp = _lin(xin, W, b.reshape(1, NH))
        uep = _lin(xein, W * a[:, 0][None, :], (b * a[:, 0]).reshape(1, NH))
        acc, se, sn, w = _sc_pass1(uep, xp, eip, nip, zacc, z1d)
        xeo = _norm_e(acc, se)
        xu = _sc_pass2(xeo, eip, nip, w, zxu)
        xo = _norm_n(xu, sn)
        return xo, xeo

    x1, xe1 = layer(xpad, xepad, W1, b1, a1)
    x2, xe2p = layer(x1, xe1, W2, b2, a2)
    out, xsg = _pool_head(x2, sgsp, cf, Wa, va, Wf, bf, Wf2, bf2, Wf3, bf3)
    return (out, xsg, out, xe2p[:E])


# speedup vs baseline: 1.2347x; 1.0527x over previous
import dataclasses
import functools

import jax
import jax.numpy as jnp
from jax import lax
from jax.experimental import pallas as pl
from jax.experimental.pallas import tpu as pltpu
from jax.experimental.pallas import tpu_sc as plsc

N = 10000
E = 5000
NNZ = 320000
D = 128
NH = 128
B = 1024
DCF = 16
NCLS = 10
LH = 2 * NH // 3

EPAD = 5120
NPAD = 10240
CH = 128
CH1 = 64
NW = 32
NNZP = 327680
CHUNKS = NNZP // CH
CHUNKS1 = NNZP // CH1
CPT = CHUNKS1 // NW
NHALF = NPAD // 2
XUP = NHALF + 128
XTR = XUP // 16
EACC = 5120
EDUM = EACC - 1

POOL_BLK = 1024
POOL_STEPS = NPAD // POOL_BLK

_PREC = jax.lax.Precision.HIGHEST
_MESH = plsc.VectorSubcoreMesh(core_axis_name="c", subcore_axis_name="s")

_SC_PARAMS = pltpu.CompilerParams()
if "needs_layout_passes" in pltpu.CompilerParams.__dataclass_fields__:
    _SC_PARAMS = dataclasses.replace(_SC_PARAMS, needs_layout_passes=False)


@functools.partial(
    pl.kernel,
    out_type=[
        jax.ShapeDtypeStruct((2, EACC, D), jnp.float32),
        jax.ShapeDtypeStruct((NW, EPAD), jnp.float32),
        jax.ShapeDtypeStruct((NW, NPAD), jnp.float32),
        jax.ShapeDtypeStruct((NNZP, 16), jnp.float32),
    ],
    mesh=_MESH,
    compiler_params=_SC_PARAMS,
    scratch_types=[
        pltpu.VMEM((CH1,), jnp.int32),
        pltpu.VMEM((CH1,), jnp.int32),
        pltpu.VMEM((CH1,), jnp.int32),
        pltpu.VMEM((CH1,), jnp.int32),
        pltpu.VMEM((CH1, D), jnp.float32),
        pltpu.VMEM((CH1, D), jnp.float32),
        pltpu.VMEM((CH1, D), jnp.float32),
        pltpu.VMEM((CH1, D), jnp.float32),
        pltpu.VMEM((CH1, D), jnp.float32),
        pltpu.VMEM((CH1, 16), jnp.float32),
        pltpu.VMEM((EPAD,), jnp.float32),
        pltpu.VMEM((NPAD,), jnp.float32),
        pltpu.VMEM_SHARED((EACC, D), jnp.float32),
        pltpu.SemaphoreType.DMA,
        pltpu.SemaphoreType.DMA,
    ],
)
def _sc_pass1(uep_hbm, xp_hbm, ei_hbm, ni_hbm, zacc_hbm, z1d_hbm,
              acc_out, se_out, sn_out, w_out,
              ei_a, ni_a, ei_b, ni_b, ue_a, xr_a, ue_b, xr_b, val_v,
              w_v, se_t, sn_t, acc_sh, sem_a, sem_b):
    c = lax.axis_index("c")
    s = lax.axis_index("s")
    wid = s * 2 + c
    first = wid * CPT
    last = first + CPT - 1

    pltpu.sync_copy(zacc_hbm, acc_sh.at[pl.ds(s * (EACC // 16), EACC // 16)])
    pltpu.sync_copy(z1d_hbm.at[pl.ds(0, EPAD)], se_t)
    pltpu.sync_copy(z1d_hbm, sn_t)
    plsc.subcore_barrier()

    lane0 = lax.iota(jnp.int32, 16) == 0

    def fetch(t, ei_v, ni_v, ue_v, xr_v, sem):
        pltpu.sync_copy(ei_hbm.at[pl.ds(t * CH1, CH1)], ei_v)
        pltpu.sync_copy(ni_hbm.at[pl.ds(t * CH1, CH1)], ni_v)
        g1 = pltpu.async_copy(uep_hbm.at[ei_v], ue_v, sem)
        g2 = pltpu.async_copy(xp_hbm.at[ni_v], xr_v, sem)
        return g1, g2

    def process(t, ei_v, ni_v, ue_v, xr_v):
        @pl.loop(0, CH1 // 16)
        def _groups(g):
            ev16 = ei_v[pl.ds(g * 16, 16)]
            nv16 = ni_v[pl.ds(g * 16, 16)]
            accs = [ue_v[g * 16 + i, pl.ds(0, 16)] * xr_v[g * 16 + i, pl.ds(0, 16)]
                    for i in range(16)]
            for j in range(1, 8):
                for i in range(16):
                    p = g * 16 + i
                    accs[i] = accs[i] + (ue_v[p, pl.ds(16 * j, 16)]
                                         * xr_v[p, pl.ds(16 * j, 16)])
            wvs = []
            for i in range(16):
                pe = jnp.sum(accs[i])
                pe = jnp.where(pe >= 0.0, pe, 0.2 * pe)
                wvs.append(jnp.exp(jnp.full((16,), pe, jnp.float32)))
            for i in range(16):
                p = g * 16 + i
                w_v[p, pl.ds(0, 16)] = wvs[i]
                for j in range(8):
                    val_v[p, pl.ds(16 * j, 16)] = wvs[i] * xr_v[p, pl.ds(16 * j, 16)]
            for i in range(16):
                eidx = jnp.full((16,), ev16[i], jnp.int32)
                nidx = jnp.full((16,), nv16[i], jnp.int32)
                plsc.addupdate_scatter(se_t, [eidx], wvs[i], mask=lane0)
                plsc.addupdate_scatter(sn_t, [nidx], wvs[i], mask=lane0)

        pltpu.sync_copy(val_v, acc_sh.at[ei_v], add=True)
        pltpu.sync_copy(w_v, w_out.at[pl.ds(t * CH1, CH1)])

    ga = fetch(first, ei_a, ni_a, ue_a, xr_a, sem_a)

    @pl.loop(0, CPT // 2)
    def _chunks(u):
        t0 = first + 2 * u
        gb = fetch(t0 + 1, ei_b, ni_b, ue_b, xr_b, sem_b)
        for g in ga:
            g.wait()
        process(t0, ei_a, ni_a, ue_a, xr_a)
        for g in gb:
            g.wait()
        ga2 = fetch(jnp.minimum(t0 + 2, last), ei_a, ni_a, ue_a, xr_a, sem_a)
        process(t0 + 1, ei_b, ni_b, ue_b, xr_b)

    for g in ga:
        g.wait()

    plsc.subcore_barrier()

    eslc = pl.ds(s * (EACC // 16), EACC // 16)
    pltpu.sync_copy(acc_sh.at[eslc], acc_out.at[c, eslc])
    pltpu.sync_copy(se_t, se_out.at[wid])
    pltpu.sync_copy(sn_t, sn_out.at[wid])


CPT2 = CHUNKS // 16


@functools.partial(
    pl.kernel,
    out_type=jax.ShapeDtypeStruct((2, XUP, D), jnp.float32),
    mesh=_MESH,
    compiler_params=_SC_PARAMS,
    scratch_types=[
        pltpu.VMEM((CH,), jnp.int32),
        pltpu.VMEM((CH,), jnp.int32),
        pltpu.VMEM((CH,), jnp.int32),
        pltpu.VMEM((CH,), jnp.int32),
        pltpu.VMEM((CH,), jnp.int32),
        pltpu.VMEM((CH, D), jnp.float32),
        pltpu.VMEM((CH, D), jnp.float32),
        pltpu.VMEM((CH, D), jnp.float32),
        pltpu.VMEM((CH, 16), jnp.float32),
        pltpu.VMEM((CH, 16), jnp.float32),
        pltpu.VMEM_SHARED((XUP, D), jnp.float32),
        pltpu.SemaphoreType.DMA,
        pltpu.SemaphoreType.DMA,
    ],
)
def _sc_pass2(xeo_hbm, ei_hbm, ni_hbm, w_hbm, zxu_hbm, xu_out,
              ei_a, ni_a, ei_b, ni_b, ni2_v, xe_a, xe_b, xval_v, w_a, w_b,
              xu_sh, sem_a, sem_b):
    c = lax.axis_index("c")
    s = lax.axis_index("s")
    offs = c * NHALF
    first = s * CPT2
    last = first + CPT2 - 1

    pltpu.sync_copy(zxu_hbm, xu_sh.at[pl.ds(s * XTR, XTR)])
    plsc.subcore_barrier()

    trash = jnp.full((16,), NHALF, jnp.int32)

    def fetch(t, ei_v, ni_v, xe_v, w_v, sem):
        pltpu.sync_copy(ei_hbm.at[pl.ds(t * CH, CH)], ei_v)
        pltpu.sync_copy(ni_hbm.at[pl.ds(t * CH, CH)], ni_v)
        g1 = pltpu.async_copy(xeo_hbm.at[ei_v], xe_v, sem)
        g2 = pltpu.async_copy(w_hbm.at[pl.ds(t * CH, CH)], w_v, sem)
        return g1, g2

    def process(ni_v, xe_v, w_v):
        @pl.loop(0, CH // 16)
        def _groups(g):
            nv16 = ni_v[pl.ds(g * 16, 16)]
            lidx = nv16 - offs
            ok = (lidx >= 0) & (lidx < NHALF)
            ni2_v[pl.ds(g * 16, 16)] = jnp.where(ok, lidx, trash)
            wvs = [w_v[g * 16 + i, pl.ds(0, 16)] for i in range(16)]
            for j in range(8):
                for i in range(16):
                    p = g * 16 + i
                    xval_v[p, pl.ds(16 * j, 16)] = wvs[i] * xe_v[p, pl.ds(16 * j, 16)]

        pltpu.sync_copy(xval_v, xu_sh.at[ni2_v], add=True)

    ga = fetch(first, ei_a, ni_a, xe_a, w_a, sem_a)

    @pl.loop(0, CPT2 // 2)
    def _chunks(u):
        t0 = first + 2 * u
        gb = fetch(t0 + 1, ei_b, ni_b, xe_b, w_b, sem_b)
        for g in ga:
            g.wait()
        process(ni_a, xe_a, w_a)
        for g in gb:
            g.wait()
        ga2 = fetch(jnp.minimum(t0 + 2, last), ei_a, ni_a, xe_a, w_a, sem_a)
        process(ni_b, xe_b, w_b)

    for g in ga:
        g.wait()

    plsc.subcore_barrier()

    nslc = pl.ds(s * XTR, XTR)
    pltpu.sync_copy(xu_sh.at[nslc], xu_out.at[c, nslc])


def _lin_body(x_ref, w_ref, b_ref, o_ref):
    o_ref[...] = (jnp.dot(x_ref[...], w_ref[...], precision=_PREC,
                          preferred_element_type=jnp.float32) + b_ref[...])


def _lin(x, w, b):
    rows = x.shape[0]
    return pl.pallas_call(
        _lin_body,
        grid=(rows // 1024,),
        in_specs=[
            pl.BlockSpec((1024, D), lambda i: (i, 0)),
            pl.BlockSpec((D, NH), lambda i: (0, 0)),
            pl.BlockSpec((1, NH), lambda i: (0, 0)),
        ],
        out_specs=pl.BlockSpec((1024, NH), lambda i: (i, 0)),
        out_shape=jax.ShapeDtypeStruct((rows, NH), jnp.float32),
    )(x, w, b)


def _norm_e_body(x0_ref, x1_ref, s_ref, o_ref):
    ssum = jnp.sum(s_ref[...], axis=0)[:, None]
    o_ref[...] = (x0_ref[0] + x1_ref[0]) / (ssum + 1e-9)


def _norm_e(acc, ssum):
    return pl.pallas_call(
        _norm_e_body,
        grid=((EACC + 1023) // 1024,),
        in_specs=[
            pl.BlockSpec((1, 1024, D), lambda i: (0, i, 0)),
            pl.BlockSpec((1, 1024, D), lambda i: (1, i, 0)),
            pl.BlockSpec((NW, 1024), lambda i: (0, i)),
        ],
        out_specs=pl.BlockSpec((1024, D), lambda i: (i, 0)),
        out_shape=jax.ShapeDtypeStruct((EACC, D), jnp.float32),
    )(acc, acc, ssum)


def _norm_n_body(x_ref, s_ref, o_ref):
    ssum = jnp.sum(s_ref[...], axis=0)[:, None]
    o_ref[...] = x_ref[0] / (ssum + 1e-9)


def _norm_n(xu, ssum):
    nblk = NHALF // 1024
    return pl.pallas_call(
        _norm_n_body,
        grid=(NPAD // 1024,),
        in_specs=[
            pl.BlockSpec((1, 1024, D), lambda i: (i // nblk, i % nblk, 0)),
            pl.BlockSpec((NW, 1024), lambda i: (0, i)),
        ],
        out_specs=pl.BlockSpec((1024, D), lambda i: (i, 0)),
        out_shape=jax.ShapeDtypeStruct((NPAD, D), jnp.float32),
    )(xu, ssum)


def _pool_head_body(x2_ref, sgs_ref, cf_ref, Wa_ref, va_ref, Wf_ref, bf_ref,
                    Wf2_ref, bf2_ref, Wf3_ref, bf3_ref,
                    out_ref, xsg_ref,
                    num_acc, den_acc, col_acc):
    j = pl.program_id(0)

    @pl.when(j == 0)
    def _init():
        num_acc[...] = jnp.zeros_like(num_acc)
        den_acc[...] = jnp.zeros_like(den_acc)
        col_acc[...] = jnp.zeros_like(col_acc)

    x2b = x2_ref[...]
    sgsb = sgs_ref[...]
    sb = jnp.dot(jnp.tanh(jnp.dot(x2b, Wa_ref[...], precision=_PREC,
                                  preferred_element_type=jnp.float32)),
                 va_ref[...], precision=_PREC,
                 preferred_element_type=jnp.float32)
    es = jnp.exp(sb)
    y2 = x2b * es
    num_acc[...] += jnp.dot(sgsb, y2, precision=_PREC,
                            preferred_element_type=jnp.float32)
    den_acc[...] += jnp.dot(sgsb, es, precision=_PREC,
                            preferred_element_type=jnp.float32)
    rowid = lax.broadcasted_iota(jnp.int32, (POOL_BLK, 1), 0) + j * POOL_BLK
    col_acc[...] += jnp.sum(jnp.where(rowid < N, x2b, 0.0), axis=0,
                            keepdims=True)

    @pl.when(j == POOL_STEPS - 1)
    def _final():
        den = den_acc[...]
        mean = col_acc[...] / N
        xsg = jnp.where(den > 0, num_acc[...] / jnp.where(den > 0, den, 1.0),
                        mean)
        xsg_ref[...] = xsg
        hcat = jnp.concatenate([xsg, cf_ref[...]], axis=1)
        h = jnp.maximum(jnp.dot(hcat, Wf_ref[...], precision=_PREC,
                                preferred_element_type=jnp.float32)
                        + bf_ref[...], 0.0)
        h = jnp.maximum(jnp.dot(h, Wf2_ref[...], precision=_PREC,
                                preferred_element_type=jnp.float32)
                        + bf2_ref[...], 0.0)
        out_ref[...] = jnp.dot(h, Wf3_ref[...], precision=_PREC,
                               preferred_element_type=jnp.float32) + bf3_ref[...]


def _pool_head(x2p, sgsp, cf, Wa, va, Wf, bf, Wf2, bf2, Wf3, bf3):
    full = lambda shape: pl.BlockSpec(shape, lambda j: (0,) * len(shape))
    out, xsg = pl.pallas_call(
        _pool_head_body,
        grid=(POOL_STEPS,),
        in_specs=[
            pl.BlockSpec((POOL_BLK, D), lambda j: (j, 0)),
            pl.BlockSpec((B, POOL_BLK), lambda j: (0, j)),
            full((B, DCF)),
            full((NH, NH)),
            full((NH, 1)),
            full((NH + DCF, LH)),
            full((LH,)),
            full((LH, LH)),
            full((LH,)),
            full((LH, NCLS)),
            full((NCLS,)),
        ],
        out_specs=[
            pl.BlockSpec((B, NCLS), lambda j: (0, 0)),
            pl.BlockSpec((B, NH), lambda j: (0, 0)),
        ],
        out_shape=[
            jax.ShapeDtypeStruct((B, NCLS), jnp.float32),
            jax.ShapeDtypeStruct((B, NH), jnp.float32),
        ],
        scratch_shapes=[
            pltpu.VMEM((B, NH), jnp.float32),
            pltpu.VMEM((B, 1), jnp.float32),
            pltpu.VMEM((1, D), jnp.float32),
        ],
    )(x2p, sgsp, cf, Wa, va, Wf, bf, Wf2, bf2, Wf3, bf3)
    return out, xsg


def kernel(x, xe, sgs, cf, W1, b1, a1, W2, b2, a2, Wa, va, Wf, bf, Wf2, bf2,
           Wf3, bf3, pair):
    f32 = jnp.float32
    xpad = jnp.zeros((NPAD, D), f32).at[:N].set(x)
    xepad = jnp.zeros((EPAD, D), f32).at[:E].set(xe)
    npad = NNZP - NNZ
    eip = jnp.concatenate([pair[0], jnp.full((npad,), EDUM, jnp.int32)])
    nip = jnp.concatenate([pair[1], jnp.full((npad,), NPAD - 1, jnp.int32)])
    sgsp = jnp.zeros((B, NPAD), f32).at[:, :N].set(sgs)
    zacc = jnp.zeros((EACC // 16, D), f32)
    z1d = jnp.zeros((NPAD,), f32)
    zxu = jnp.zeros((XTR, D), f32)

    def layer(xin, xein, W, b, a):
        xp = _lin(xin, W, b.reshape(1, NH))
        uep = _lin(xein, W * a[:, 0][None, :], (b * a[:, 0]).reshape(1, NH))
        acc, se, sn, w = _sc_pass1(uep, xp, eip, nip, zacc, z1d)
        xeo = _norm_e(acc, se)
        xu = _sc_pass2(xeo, eip, nip, w, zxu)
        xo = _norm_n(xu, sn)
        return xo, xeo

    x1, xe1 = layer(xpad, xepad, W1, b1, a1)
    x2, xe2p = layer(x1, xe1, W2, b2, a2)
    out, xsg = _pool_head(x2, sgsp, cf, Wa, va, Wf, bf, Wf2, bf2, Wf3, bf3)
    return (out, xsg, out, xe2p[:E])


# speedup vs baseline: 1.2773x; 1.0345x over previous
import dataclasses
import functools

import jax
import jax.numpy as jnp
from jax import lax
from jax.experimental import pallas as pl
from jax.experimental.pallas import tpu as pltpu
from jax.experimental.pallas import tpu_sc as plsc

N = 10000
E = 5000
NNZ = 320000
D = 128
NH = 128
B = 1024
DCF = 16
NCLS = 10
LH = 2 * NH // 3

EPAD = 5120
NPAD = 10240
CH = 128
CH1 = 64
NW = 32
NNZP = 327680
CHUNKS = NNZP // CH
CHUNKS1 = NNZP // CH1
CPT = CHUNKS1 // NW
SEG1 = 32
NSEG1 = CPT // SEG1
NHALF = NPAD // 2
XUP = NHALF + 8
XTR = XUP // 16
EACC = 5008
EDUM = EACC - 1

POOL_BLK = 1024
POOL_STEPS = NPAD // POOL_BLK

_PREC = jax.lax.Precision.HIGHEST
_MESH = plsc.VectorSubcoreMesh(core_axis_name="c", subcore_axis_name="s")

_SC_PARAMS = pltpu.CompilerParams()
if "needs_layout_passes" in pltpu.CompilerParams.__dataclass_fields__:
    _SC_PARAMS = dataclasses.replace(_SC_PARAMS, needs_layout_passes=False)


@functools.partial(
    pl.kernel,
    out_type=[
        jax.ShapeDtypeStruct((2, EACC, D), jnp.float32),
        jax.ShapeDtypeStruct((NW, EACC), jnp.float32),
        jax.ShapeDtypeStruct((NW, NPAD), jnp.float32),
        jax.ShapeDtypeStruct((NNZP, 16), jnp.float32),
    ],
    mesh=_MESH,
    compiler_params=_SC_PARAMS,
    scratch_types=[
        pltpu.VMEM((SEG1 * CH1,), jnp.int32),
        pltpu.VMEM((SEG1 * CH1,), jnp.int32),
        pltpu.VMEM((CH1,), jnp.int32),
        pltpu.VMEM((CH1, D), jnp.float32),
        pltpu.VMEM((CH1, D), jnp.float32),
        pltpu.VMEM((CH1, D), jnp.float32),
        pltpu.VMEM((CH1, D), jnp.float32),
        pltpu.VMEM((CH1, D), jnp.float32),
        pltpu.VMEM((CH1, 16), jnp.float32),
        pltpu.VMEM((EACC,), jnp.float32),
        pltpu.VMEM((NPAD,), jnp.float32),
        pltpu.VMEM_SHARED((EACC, D), jnp.float32),
        pltpu.SemaphoreType.DMA,
        pltpu.SemaphoreType.DMA,
    ],
)
def _sc_pass1(uep_hbm, xp_hbm, ei_hbm, ni_hbm, zacc_hbm, z1d_hbm,
              acc_out, se_out, sn_out, w_out,
              ei_q, ni_q, ei_s, ue_a, xr_a, ue_b, xr_b, val_v,
              w_v, se_t, sn_t, acc_sh, sem_a, sem_b):
    c = lax.axis_index("c")
    s = lax.axis_index("s")
    wid = s * 2 + c
    first = wid * CPT

    @pl.when(s == 0)
    def _zero():
        pltpu.sync_copy(zacc_hbm, acc_sh)

    pltpu.sync_copy(z1d_hbm.at[pl.ds(0, EACC)], se_t)
    pltpu.sync_copy(z1d_hbm, sn_t)
    plsc.subcore_barrier()

    lane0 = lax.iota(jnp.int32, 16) == 0

    def fetch(lb, ue_v, xr_v, sem):
        g1 = pltpu.async_copy(uep_hbm.at[ei_q.at[pl.ds(lb, CH1)]], ue_v, sem)
        g2 = pltpu.async_copy(xp_hbm.at[ni_q.at[pl.ds(lb, CH1)]], xr_v, sem)
        return g1, g2

    def process(t, lb, ue_v, xr_v):
        @pl.loop(0, CH1 // 16)
        def _groups(g):
            ev16 = ei_q[pl.ds(lb + g * 16, 16)]
            nv16 = ni_q[pl.ds(lb + g * 16, 16)]
            accs = [ue_v[g * 16 + i, pl.ds(0, 16)] * xr_v[g * 16 + i, pl.ds(0, 16)]
                    for i in range(16)]
            for j in range(1, 8):
                for i in range(16):
                    p = g * 16 + i
                    accs[i] = accs[i] + (ue_v[p, pl.ds(16 * j, 16)]
                                         * xr_v[p, pl.ds(16 * j, 16)])
            wvs = []
            for i in range(16):
                pe = jnp.sum(accs[i])
                pe = jnp.where(pe >= 0.0, pe, 0.2 * pe)
                wvs.append(jnp.exp(jnp.full((16,), pe, jnp.float32)))
            for i in range(16):
                p = g * 16 + i
                w_v[p, pl.ds(0, 16)] = wvs[i]
                for j in range(8):
                    val_v[p, pl.ds(16 * j, 16)] = wvs[i] * xr_v[p, pl.ds(16 * j, 16)]
            for i in range(16):
                eidx = jnp.full((16,), ev16[i], jnp.int32)
                nidx = jnp.full((16,), nv16[i], jnp.int32)
                plsc.addupdate_scatter(se_t, [eidx], wvs[i], mask=lane0)
                plsc.addupdate_scatter(sn_t, [nidx], wvs[i], mask=lane0)

        for k in range(CH1 // 16):
            ei_s[pl.ds(16 * k, 16)] = ei_q[pl.ds(lb + 16 * k, 16)]
        pltpu.sync_copy(val_v, acc_sh.at[ei_s], add=True)
        pltpu.sync_copy(w_v, w_out.at[pl.ds(t * CH1, CH1)])

    @pl.loop(0, NSEG1)
    def _segs(sg):
        sbase = (first + sg * SEG1) * CH1
        pltpu.sync_copy(ei_hbm.at[pl.ds(sbase, SEG1 * CH1)], ei_q)
        pltpu.sync_copy(ni_hbm.at[pl.ds(sbase, SEG1 * CH1)], ni_q)
        ga = fetch(0, ue_a, xr_a, sem_a)

        @pl.loop(0, SEG1 // 2)
        def _chunks(u):
            t0 = first + sg * SEG1 + 2 * u
            lb0 = 2 * u * CH1
            gb = fetch(lb0 + CH1, ue_b, xr_b, sem_b)
            for g in ga:
                g.wait()
            process(t0, lb0, ue_a, xr_a)
            for g in gb:
                g.wait()
            ga2 = fetch(jnp.minimum(lb0 + 2 * CH1, (SEG1 - 1) * CH1),
                        ue_a, xr_a, sem_a)
            process(t0 + 1, lb0 + CH1, ue_b, xr_b)

        for g in ga:
            g.wait()

    plsc.subcore_barrier()

    @pl.when(s == 0)
    def _flush():
        pltpu.sync_copy(acc_sh, acc_out.at[c])

    pltpu.sync_copy(se_t, se_out.at[wid])
    pltpu.sync_copy(sn_t, sn_out.at[wid])


CPT2 = CHUNKS // 16
SEG2 = 20
NSEG2 = CPT2 // SEG2


@functools.partial(
    pl.kernel,
    out_type=jax.ShapeDtypeStruct((2, XUP, D), jnp.float32),
    mesh=_MESH,
    compiler_params=_SC_PARAMS,
    scratch_types=[
        pltpu.VMEM((SEG2 * CH,), jnp.int32),
        pltpu.VMEM((SEG2 * CH,), jnp.int32),
        pltpu.VMEM((CH,), jnp.int32),
        pltpu.VMEM((CH, D), jnp.float32),
        pltpu.VMEM((CH, D), jnp.float32),
        pltpu.VMEM((CH, D), jnp.float32),
        pltpu.VMEM((CH, 16), jnp.float32),
        pltpu.VMEM((CH, 16), jnp.float32),
        pltpu.VMEM_SHARED((XUP, D), jnp.float32),
        pltpu.SemaphoreType.DMA,
        pltpu.SemaphoreType.DMA,
    ],
)
def _sc_pass2(xeo_hbm, ei_hbm, ni_hbm, w_hbm, zxu_hbm, xu_out,
              ei_q, ni_q, ni2_v, xe_a, xe_b, xval_v, w_a, w_b,
              xu_sh, sem_a, sem_b):
    c = lax.axis_index("c")
    s = lax.axis_index("s")
    offs = c * NHALF
    first = s * CPT2

    @pl.when(s == 0)
    def _zero():
        pltpu.sync_copy(zxu_hbm, xu_sh)

    plsc.subcore_barrier()

    trash = jnp.full((16,), NHALF, jnp.int32)

    def fetch(t, lb, xe_v, w_v, sem):
        g1 = pltpu.async_copy(xeo_hbm.at[ei_q.at[pl.ds(lb, CH)]], xe_v, sem)
        g2 = pltpu.async_copy(w_hbm.at[pl.ds(t * CH, CH)], w_v, sem)
        return g1, g2

    def process(lb, xe_v, w_v):
        @pl.loop(0, CH // 16)
        def _groups(g):
            nv16 = ni_q[pl.ds(lb + g * 16, 16)]
            lidx = nv16 - offs
            ok = (lidx >= 0) & (lidx < NHALF)
            ni2_v[pl.ds(g * 16, 16)] = jnp.where(ok, lidx, trash)
            wvs = [w_v[g * 16 + i, pl.ds(0, 16)] for i in range(16)]
            for j in range(8):
                for i in range(16):
                    p = g * 16 + i
                    xval_v[p, pl.ds(16 * j, 16)] = wvs[i] * xe_v[p, pl.ds(16 * j, 16)]

        pltpu.sync_copy(xval_v, xu_sh.at[ni2_v], add=True)

    @pl.loop(0, NSEG2)
    def _segs(sg):
        sbase = (first + sg * SEG2) * CH
        pltpu.sync_copy(ei_hbm.at[pl.ds(sbase, SEG2 * CH)], ei_q)
        pltpu.sync_copy(ni_hbm.at[pl.ds(sbase, SEG2 * CH)], ni_q)
        ga = fetch(first + sg * SEG2, 0, xe_a, w_a, sem_a)

        @pl.loop(0, SEG2 // 2)
        def _chunks(u):
            t0 = first + sg * SEG2 + 2 * u
            lb0 = 2 * u * CH
            gb = fetch(t0 + 1, lb0 + CH, xe_b, w_b, sem_b)
            for g in ga:
                g.wait()
            process(lb0, xe_a, w_a)
            for g in gb:
                g.wait()
            ga2 = fetch(jnp.minimum(t0 + 2, first + sg * SEG2 + SEG2 - 1),
                        jnp.minimum(lb0 + 2 * CH, (SEG2 - 1) * CH),
                        xe_a, w_a, sem_a)
            process(lb0 + CH, xe_b, w_b)

        for g in ga:
            g.wait()

    plsc.subcore_barrier()

    @pl.when(s == 0)
    def _flush():
        pltpu.sync_copy(xu_sh, xu_out.at[c])


def _lin_body(x_ref, w_ref, b_ref, o_ref):
    o_ref[...] = (jnp.dot(x_ref[...], w_ref[...], precision=_PREC,
                          preferred_element_type=jnp.float32) + b_ref[...])


def _lin(x, w, b):
    rows = x.shape[0]
    return pl.pallas_call(
        _lin_body,
        grid=(rows // 1024,),
        in_specs=[
            pl.BlockSpec((1024, D), lambda i: (i, 0)),
            pl.BlockSpec((D, NH), lambda i: (0, 0)),
            pl.BlockSpec((1, NH), lambda i: (0, 0)),
        ],
        out_specs=pl.BlockSpec((1024, NH), lambda i: (i, 0)),
        out_shape=jax.ShapeDtypeStruct((rows, NH), jnp.float32),
    )(x, w, b)


def _norm_e_body(x0_ref, x1_ref, s_ref, o_ref):
    ssum = jnp.sum(s_ref[...], axis=0)[:, None]
    o_ref[...] = (x0_ref[0] + x1_ref[0]) / (ssum + 1e-9)


def _norm_e(acc, ssum):
    return pl.pallas_call(
        _norm_e_body,
        grid=((EACC + 1023) // 1024,),
        in_specs=[
            pl.BlockSpec((1, 1024, D), lambda i: (0, i, 0)),
            pl.BlockSpec((1, 1024, D), lambda i: (1, i, 0)),
            pl.BlockSpec((NW, 1024), lambda i: (0, i)),
        ],
        out_specs=pl.BlockSpec((1024, D), lambda i: (i, 0)),
        out_shape=jax.ShapeDtypeStruct((EACC, D), jnp.float32),
    )(acc, acc, ssum)


def _norm_n_body(x_ref, s_ref, o_ref):
    ssum = jnp.sum(s_ref[...], axis=0)[:, None]
    o_ref[...] = x_ref[0] / (ssum + 1e-9)


def _norm_n(xu, ssum):
    nblk = NHALF // 1024
    return pl.pallas_call(
        _norm_n_body,
        grid=(NPAD // 1024,),
        in_specs=[
            pl.BlockSpec((1, 1024, D), lambda i: (i // nblk, i % nblk, 0)),
            pl.BlockSpec((NW, 1024), lambda i: (0, i)),
        ],
        out_specs=pl.BlockSpec((1024, D), lambda i: (i, 0)),
        out_shape=jax.ShapeDtypeStruct((NPAD, D), jnp.float32),
    )(xu, ssum)


def _pool_head_body(x2_ref, sgs_ref, cf_ref, Wa_ref, va_ref, Wf_ref, bf_ref,
                    Wf2_ref, bf2_ref, Wf3_ref, bf3_ref,
                    out_ref, xsg_ref,
                    num_acc, den_acc, col_acc):
    j = pl.program_id(0)

    @pl.when(j == 0)
    def _init():
        num_acc[...] = jnp.zeros_like(num_acc)
        den_acc[...] = jnp.zeros_like(den_acc)
        col_acc[...] = jnp.zeros_like(col_acc)

    x2b = x2_ref[...]
    sgsb = sgs_ref[...]
    sb = jnp.dot(jnp.tanh(jnp.dot(x2b, Wa_ref[...], precision=_PREC,
                                  preferred_element_type=jnp.float32)),
                 va_ref[...], precision=_PREC,
                 preferred_element_type=jnp.float32)
    es = jnp.exp(sb)
    y2 = x2b * es
    num_acc[...] += jnp.dot(sgsb, y2, precision=_PREC,
                            preferred_element_type=jnp.float32)
    den_acc[...] += jnp.dot(sgsb, es, precision=_PREC,
                            preferred_element_type=jnp.float32)
    rowid = lax.broadcasted_iota(jnp.int32, (POOL_BLK, 1), 0) + j * POOL_BLK
    col_acc[...] += jnp.sum(jnp.where(rowid < N, x2b, 0.0), axis=0,
                            keepdims=True)

    @pl.when(j == POOL_STEPS - 1)
    def _final():
        den = den_acc[...]
        mean = col_acc[...] / N
        xsg = jnp.where(den > 0, num_acc[...] / jnp.where(den > 0, den, 1.0),
                        mean)
        xsg_ref[...] = xsg
        hcat = jnp.concatenate([xsg, cf_ref[...]], axis=1)
        h = jnp.maximum(jnp.dot(hcat, Wf_ref[...], precision=_PREC,
                                preferred_element_type=jnp.float32)
                        + bf_ref[...], 0.0)
        h = jnp.maximum(jnp.dot(h, Wf2_ref[...], precision=_PREC,
                                preferred_element_type=jnp.float32)
                        + bf2_ref[...], 0.0)
        out_ref[...] = jnp.dot(h, Wf3_ref[...], precision=_PREC,
                               preferred_element_type=jnp.float32) + bf3_ref[...]


def _pool_head(x2p, sgsp, cf, Wa, va, Wf, bf, Wf2, bf2, Wf3, bf3):
    full = lambda shape: pl.BlockSpec(shape, lambda j: (0,) * len(shape))
    out, xsg = pl.pallas_call(
        _pool_head_body,
        grid=(POOL_STEPS,),
        in_specs=[
            pl.BlockSpec((POOL_BLK, D), lambda j: (j, 0)),
            pl.BlockSpec((B, POOL_BLK), lambda j: (0, j)),
            full((B, DCF)),
            full((NH, NH)),
            full((NH, 1)),
            full((NH + DCF, LH)),
            full((LH,)),
            full((LH, LH)),
            full((LH,)),
            full((LH, NCLS)),
            full((NCLS,)),
        ],
        out_specs=[
            pl.BlockSpec((B, NCLS), lambda j: (0, 0)),
            pl.BlockSpec((B, NH), lambda j: (0, 0)),
        ],
        out_shape=[
            jax.ShapeDtypeStruct((B, NCLS), jnp.float32),
            jax.ShapeDtypeStruct((B, NH), jnp.float32),
        ],
        scratch_shapes=[
            pltpu.VMEM((B, NH), jnp.float32),
            pltpu.VMEM((B, 1), jnp.float32),
            pltpu.VMEM((1, D), jnp.float32),
        ],
    )(x2p, sgsp, cf, Wa, va, Wf, bf, Wf2, bf2, Wf3, bf3)
    return out, xsg


def kernel(x, xe, sgs, cf, W1, b1, a1, W2, b2, a2, Wa, va, Wf, bf, Wf2, bf2,
           Wf3, bf3, pair):
    f32 = jnp.float32
    xpad = jnp.zeros((NPAD, D), f32).at[:N].set(x)
    xepad = jnp.zeros((EPAD, D), f32).at[:E].set(xe)
    npad = NNZP - NNZ
    eip = jnp.concatenate([pair[0], jnp.full((npad,), EDUM, jnp.int32)])
    nip = jnp.concatenate([pair[1], jnp.full((npad,), NPAD - 1, jnp.int32)])
    sgsp = jnp.zeros((B, NPAD), f32).at[:, :N].set(sgs)
    zacc = jnp.zeros((EACC, D), f32)
    z1d = jnp.zeros((NPAD,), f32)
    zxu = jnp.zeros((XUP, D), f32)

    def layer(xin, xein, W, b, a):
        xp = _lin(xin, W, b.reshape(1, NH))
        uep = _lin(xein, W * a[:, 0][None, :], (b * a[:, 0]).reshape(1, NH))
        acc, se, sn, w = _sc_pass1(uep, xp, eip, nip, zacc, z1d)
        xeo = _norm_e(acc, se)
        xu = _sc_pass2(xeo, eip, nip, w, zxu)
        xo = _norm_n(xu, sn)
        return xo, xeo

    x1, xe1 = layer(xpad, xepad, W1, b1, a1)
    x2, xe2p = layer(x1, xe1, W2, b2, a2)
    out, xsg = _pool_head(x2, sgsp, cf, Wa, va, Wf, bf, Wf2, bf2, Wf3, bf3)
    return (out, xsg, out, xe2p[:E])
